# Initial kernel scaffold; baseline (speedup 1.0000x reference)
import functools

import jax
import jax.numpy as jnp
from jax import lax
from jax.experimental import pallas as pl
from jax.experimental.pallas import tpu as pltpu
from jax.experimental.pallas import tpu_sc as plsc

_BATCH = 16384
_EMB = 32
_EXTRA = 42


def _make_sc_gather(batch, emb_dim):
    info = plsc.get_sparse_core_info()
    nw = info.num_cores * info.num_subcores
    assert batch % (8 * nw) == 0
    per_w = batch // nw
    mesh = plsc.VectorSubcoreMesh(core_axis_name="c", subcore_axis_name="s")

    @functools.partial(
        pl.kernel,
        out_type=(
            jax.ShapeDtypeStruct((batch, emb_dim), jnp.float32),
            jax.ShapeDtypeStruct((batch, emb_dim), jnp.float32),
        ),
        mesh=mesh,
        scratch_types=[
            pltpu.VMEM((per_w,), jnp.int32),
            pltpu.VMEM((per_w,), jnp.int32),
            pltpu.VMEM((per_w, emb_dim), jnp.float32),
            pltpu.VMEM((per_w, emb_dim), jnp.float32),
            pltpu.SemaphoreType.DMA,
            pltpu.SemaphoreType.DMA,
        ],
    )
    def sc_gather(table_hbm, ia_hbm, ib_hbm, ea_hbm, eb_hbm,
                  ia_v, ib_v, ra_v, rb_v, sem_a, sem_b):
        wid = lax.axis_index("s") * info.num_cores + lax.axis_index("c")
        base = wid * per_w
        pltpu.sync_copy(ia_hbm.at[pl.ds(base, per_w)], ia_v)
        pltpu.sync_copy(ib_hbm.at[pl.ds(base, per_w)], ib_v)
        cp_a = pltpu.async_copy(table_hbm.at[ia_v], ra_v, sem_a)
        cp_b = pltpu.async_copy(table_hbm.at[ib_v], rb_v, sem_b)
        cp_a.wait()
        pltpu.sync_copy(ra_v, ea_hbm.at[pl.ds(base, per_w)])
        cp_b.wait()
        pltpu.sync_copy(rb_v, eb_hbm.at[pl.ds(base, per_w)])

    return sc_gather


_sc_gather = _make_sc_gather(_BATCH, _EMB)


def _mlp_body(ea, eb, ft, w1, b1, w2, b2, w3, b3, w4, b4, out):
    x = jnp.concatenate([ea[...], eb[...], ft[...]], axis=1)
    h = jnp.maximum(jnp.dot(x, w1[...], preferred_element_type=jnp.float32)
                    + b1[...], 0.0)
    h = jnp.maximum(jnp.dot(h, w2[...], preferred_element_type=jnp.float32)
                    + b2[...], 0.0)
    h = jnp.maximum(jnp.dot(h, w3[...], preferred_element_type=jnp.float32)
                    + b3[...], 0.0)
    z = jnp.dot(h, w4[...], preferred_element_type=jnp.float32) + b4[...]
    out[...] = jax.nn.sigmoid(z)


def _mlp(ea, eb, ft, w1t, b1, w2t, b2, w3t, b3, w4t, b4, blk=2048):
    batch = ea.shape[0]
    grid = (batch // blk,)
    full = lambda a: pl.BlockSpec(a.shape, lambda i: (0, 0))
    return pl.pallas_call(
        _mlp_body,
        grid=grid,
        in_specs=[
            pl.BlockSpec((blk, ea.shape[1]), lambda i: (i, 0)),
            pl.BlockSpec((blk, eb.shape[1]), lambda i: (i, 0)),
            pl.BlockSpec((blk, ft.shape[1]), lambda i: (i, 0)),
            full(w1t), full(b1), full(w2t), full(b2),
            full(w3t), full(b3), full(w4t), full(b4),
        ],
        out_specs=pl.BlockSpec((blk, 1), lambda i: (i, 0)),
        out_shape=jax.ShapeDtypeStruct((batch, 1), jnp.float32),
    )(ea, eb, ft, w1t, b1, w2t, b2, w3t, b3, w4t, b4)


def kernel(airport_a, airport_b, features, table,
           W1, b1, W2, b2, W3, b3, W4, b4):
    ia = airport_a.astype(jnp.int32)
    ib = airport_b.astype(jnp.int32)
    emb_a, emb_b = _sc_gather(table, ia, ib)
    out = _mlp(emb_a, emb_b, features,
               W1.T, b1.reshape(1, -1),
               W2.T, b2.reshape(1, -1),
               W3.T, b3.reshape(1, -1),
               W4.T, b4.reshape(1, -1))
    return out[:, 0]


# speedup vs baseline: 1.5603x; 1.5603x over previous
import functools

import jax
import jax.numpy as jnp
from jax import lax
from jax.experimental import pallas as pl
from jax.experimental.pallas import tpu as pltpu
from jax.experimental.pallas import tpu_sc as plsc

_BATCH = 16384
_EMB = 32
_EXTRA = 42


def _make_sc_gather(batch, emb_dim):
    info = plsc.get_sparse_core_info()
    nw = info.num_cores * info.num_subcores
    assert batch % (8 * nw) == 0
    per_w = batch // nw
    mesh = plsc.VectorSubcoreMesh(core_axis_name="c", subcore_axis_name="s")

    @functools.partial(
        pl.kernel,
        out_type=(
            jax.ShapeDtypeStruct((batch, emb_dim), jnp.float32),
            jax.ShapeDtypeStruct((batch, emb_dim), jnp.float32),
        ),
        mesh=mesh,
        compiler_params=pltpu.CompilerParams(use_tc_tiling_on_sc=False),
        scratch_types=[
            pltpu.VMEM((per_w,), jnp.int32),
            pltpu.VMEM((per_w,), jnp.int32),
            pltpu.VMEM((per_w, emb_dim), jnp.float32),
            pltpu.VMEM((per_w, emb_dim), jnp.float32),
            pltpu.SemaphoreType.DMA,
            pltpu.SemaphoreType.DMA,
        ],
    )
    def sc_gather(table_hbm, ia_hbm, ib_hbm, ea_hbm, eb_hbm,
                  ia_v, ib_v, ra_v, rb_v, sem_a, sem_b):
        wid = lax.axis_index("s") * info.num_cores + lax.axis_index("c")
        base = wid * per_w
        pltpu.sync_copy(ia_hbm.at[pl.ds(base, per_w)], ia_v)
        pltpu.sync_copy(ib_hbm.at[pl.ds(base, per_w)], ib_v)
        cp_a = pltpu.async_copy(table_hbm.at[ia_v], ra_v, sem_a)
        cp_b = pltpu.async_copy(table_hbm.at[ib_v], rb_v, sem_b)
        cp_a.wait()
        pltpu.sync_copy(ra_v, ea_hbm.at[pl.ds(base, per_w)])
        cp_b.wait()
        pltpu.sync_copy(rb_v, eb_hbm.at[pl.ds(base, per_w)])

    return sc_gather


_sc_gather = _make_sc_gather(_BATCH, _EMB)


def _mlp_body(ea, eb, ft, w1, b1, w2, b2, w3, b3, w4, b4, out):
    x = jnp.concatenate([ea[...], eb[...], ft[...]], axis=1)
    h = jnp.maximum(jnp.dot(x, w1[...], preferred_element_type=jnp.float32)
                    + b1[...], 0.0)
    h = jnp.maximum(jnp.dot(h, w2[...], preferred_element_type=jnp.float32)
                    + b2[...], 0.0)
    h = jnp.maximum(jnp.dot(h, w3[...], preferred_element_type=jnp.float32)
                    + b3[...], 0.0)
    z = jnp.dot(h, w4[...], preferred_element_type=jnp.float32) + b4[...]
    out[...] = jax.nn.sigmoid(z)


def _mlp(ea, eb, ft, w1t, b1, w2t, b2, w3t, b3, w4t, b4, blk=2048):
    batch = ea.shape[0]
    grid = (batch // blk,)
    full = lambda a: pl.BlockSpec(a.shape, lambda i: (0, 0))
    return pl.pallas_call(
        _mlp_body,
        grid=grid,
        in_specs=[
            pl.BlockSpec((blk, ea.shape[1]), lambda i: (i, 0)),
            pl.BlockSpec((blk, eb.shape[1]), lambda i: (i, 0)),
            pl.BlockSpec((blk, ft.shape[1]), lambda i: (i, 0)),
            full(w1t), full(b1), full(w2t), full(b2),
            full(w3t), full(b3), full(w4t), full(b4),
        ],
        out_specs=pl.BlockSpec((blk, 1), lambda i: (i, 0)),
        out_shape=jax.ShapeDtypeStruct((batch, 1), jnp.float32),
    )(ea, eb, ft, w1t, b1, w2t, b2, w3t, b3, w4t, b4)


def kernel(airport_a, airport_b, features, table,
           W1, b1, W2, b2, W3, b3, W4, b4):
    ia = airport_a.astype(jnp.int32)
    ib = airport_b.astype(jnp.int32)
    emb_a, emb_b = _sc_gather(table, ia, ib)
    out = _mlp(emb_a, emb_b, features,
               W1.T, b1.reshape(1, -1),
               W2.T, b2.reshape(1, -1),
               W3.T, b3.reshape(1, -1),
               W4.T, b4.reshape(1, -1))
    return out[:, 0]
